# baseline (speedup 1.0000x reference)
import functools

import jax
import jax.numpy as jnp
from jax import lax
from jax.experimental import pallas as pl
from jax.experimental.pallas import tpu as pltpu
from jax.experimental.pallas import tpu_sc as plsc

N_NODES = 10000
N_EDGES = 320000
D = 128
N_CLASSES = 10
N_GRAPHS = 64

NC = 2
NS = 16
NW = NC * NS
CHUNK = 50
EPW = N_EDGES // NW
NCHUNK = EPW // CHUNK
NBUF = 4
PHASES = 5
PCH = NCHUNK // PHASES
N_PAD = 10240
RPT = N_PAD // NS
ZROWS = 40
ZCOPIES = RPT // ZROWS
WROWS = 128
WCOPIES = RPT // WROWS

BLK = 1000
NBLK = N_NODES // BLK


def _sc_agg_body(src_hbm, dst_hbm, x_hbm, out_hbm,
                 idx_s, idx_d, buf0, buf1, buf2, buf3, agg_sh,
                 sg0, sg1, sg2, sg3, ss0, ss1, ss2, ss3):
    c = lax.axis_index("c")
    s = lax.axis_index("s")
    wid = s * NC + c
    bufs = (buf0, buf1, buf2, buf3)
    sgs = (sg0, sg1, sg2, sg3)
    sss = (ss0, ss1, ss2, ss3)

    def _zero_row(i, _):
        def _zero_lane(j, _):
            buf0[i, pl.ds(j * 16, 16)] = jnp.zeros((16,), jnp.float32)
            return 0
        return lax.fori_loop(0, D // 16, _zero_lane, 0)
    lax.fori_loop(0, ZROWS, _zero_row, 0)
    zsrc = buf0.at[pl.ds(0, ZROWS)]
    for r in range(ZCOPIES):
        pltpu.sync_copy(zsrc, agg_sh.at[pl.ds(s * RPT + r * ZROWS, ZROWS)])
    plsc.subcore_barrier()

    for p in range(PHASES):
        base = wid * NCHUNK + p * PCH
        pltpu.sync_copy(src_hbm.at[pl.ds(base, PCH)], idx_s)
        pltpu.sync_copy(dst_hbm.at[pl.ds(base, PCH)], idx_d)
        pltpu.async_copy(x_hbm.at[idx_s.at[0]], bufs[0], sgs[0])
        pltpu.async_copy(x_hbm.at[idx_s.at[1]], bufs[1], sgs[1])

        def _round(k, _):
            for b in range(NBUF):
                j = NBUF * k + b
                pltpu.make_async_copy(x_hbm.at[idx_s.at[j]],
                                      bufs[b], sgs[b]).wait()
                pltpu.async_copy(bufs[b], agg_sh.at[idx_d.at[j]],
                                 sss[b], add=True)
                b2 = (b + 2) % NBUF

                @pl.when(j >= 2)
                def _():
                    pltpu.make_async_copy(bufs[b2],
                                          agg_sh.at[idx_d.at[j - 2]],
                                          sss[b2]).wait()

                @pl.when(j + 2 < PCH)
                def _():
                    pltpu.async_copy(x_hbm.at[idx_s.at[j + 2]],
                                     bufs[b2], sgs[b2])
            return 0
        lax.fori_loop(0, PCH // NBUF, _round, 0)
        pltpu.make_async_copy(bufs[2], agg_sh.at[idx_d.at[PCH - 2]],
                              sss[2]).wait()
        pltpu.make_async_copy(bufs[3], agg_sh.at[idx_d.at[PCH - 1]],
                              sss[3]).wait()

    plsc.subcore_barrier()
    for r in range(WCOPIES):
        off = s * RPT + r * WROWS
        pltpu.sync_copy(agg_sh.at[pl.ds(off, WROWS)],
                        out_hbm.at[c, pl.ds(off, WROWS)])


_sc_agg = pl.kernel(
    _sc_agg_body,
    out_type=jax.ShapeDtypeStruct((NC, N_PAD, D), jnp.float32),
    mesh=plsc.VectorSubcoreMesh(core_axis_name="c", subcore_axis_name="s",
                                num_cores=NC, num_subcores=NS),
    scratch_types=[
        pltpu.VMEM((PCH, CHUNK), jnp.int32),
        pltpu.VMEM((PCH, CHUNK), jnp.int32),
        pltpu.VMEM((CHUNK, D), jnp.float32),
        pltpu.VMEM((CHUNK, D), jnp.float32),
        pltpu.VMEM((CHUNK, D), jnp.float32),
        pltpu.VMEM((CHUNK, D), jnp.float32),
        pltpu.VMEM_SHARED((N_PAD, D), jnp.float32),
        pltpu.SemaphoreType.DMA,
        pltpu.SemaphoreType.DMA,
        pltpu.SemaphoreType.DMA,
        pltpu.SemaphoreType.DMA,
        pltpu.SemaphoreType.DMA,
        pltpu.SemaphoreType.DMA,
        pltpu.SemaphoreType.DMA,
        pltpu.SemaphoreType.DMA,
    ],
)


def _dense1_body(a0, a1, x, wrel, b, wroot, h_ref):
    agg = a0[...] + a1[...]
    h = jnp.dot(agg, wrel[...].T, preferred_element_type=jnp.float32)
    h = h + jnp.dot(x[...], wroot[...].T, preferred_element_type=jnp.float32)
    h_ref[...] = jnp.maximum(h + b[...], 0.0)


_dense1 = pl.pallas_call(
    _dense1_body,
    grid=(NBLK,),
    in_specs=[
        pl.BlockSpec((BLK, D), lambda i: (i, 0)),
        pl.BlockSpec((BLK, D), lambda i: (i, 0)),
        pl.BlockSpec((BLK, D), lambda i: (i, 0)),
        pl.BlockSpec((D, D), lambda i: (0, 0)),
        pl.BlockSpec((1, D), lambda i: (0, 0)),
        pl.BlockSpec((D, D), lambda i: (0, 0)),
    ],
    out_specs=pl.BlockSpec((BLK, D), lambda i: (i, 0)),
    out_shape=jax.ShapeDtypeStruct((N_NODES, D), jnp.float32),
)


def _dense2_body(a0, a1, h1, wrel, b, wroot, bat, wlin, blin,
                 out_ref, pool_acc, cnt_acc):
    i = pl.program_id(0)
    agg = a0[...] + a1[...]
    h = jnp.dot(agg, wrel[...].T, preferred_element_type=jnp.float32)
    h = h + jnp.dot(h1[...], wroot[...].T, preferred_element_type=jnp.float32)
    h = jnp.maximum(h + b[...], 0.0)

    seg = bat[...].reshape(1, BLK)
    gid = lax.broadcasted_iota(jnp.int32, (N_GRAPHS, BLK), 0)
    onehot = (seg == gid).astype(jnp.float32)

    @pl.when(i == 0)
    def _():
        pool_acc[...] = jnp.zeros_like(pool_acc)
        cnt_acc[...] = jnp.zeros_like(cnt_acc)

    pool_acc[...] += jnp.dot(onehot, h, preferred_element_type=jnp.float32)
    cnt_acc[...] += jnp.sum(onehot, axis=1, keepdims=True)

    @pl.when(i == pl.num_programs(0) - 1)
    def _():
        pooled = pool_acc[...] / jnp.maximum(cnt_acc[...], 1.0)
        out_ref[...] = (jnp.dot(pooled, wlin[...].T,
                                preferred_element_type=jnp.float32)
                        + blin[...])


_dense2 = pl.pallas_call(
    _dense2_body,
    grid=(NBLK,),
    in_specs=[
        pl.BlockSpec((BLK, D), lambda i: (i, 0)),
        pl.BlockSpec((BLK, D), lambda i: (i, 0)),
        pl.BlockSpec((BLK, D), lambda i: (i, 0)),
        pl.BlockSpec((D, D), lambda i: (0, 0)),
        pl.BlockSpec((1, D), lambda i: (0, 0)),
        pl.BlockSpec((D, D), lambda i: (0, 0)),
        pl.BlockSpec((1, 1, BLK), lambda i: (i, 0, 0)),
        pl.BlockSpec((N_CLASSES, D), lambda i: (0, 0)),
        pl.BlockSpec((1, N_CLASSES), lambda i: (0, 0)),
    ],
    out_specs=pl.BlockSpec((N_GRAPHS, N_CLASSES), lambda i: (0, 0)),
    out_shape=jax.ShapeDtypeStruct((N_GRAPHS, N_CLASSES), jnp.float32),
    scratch_shapes=[
        pltpu.VMEM((N_GRAPHS, D), jnp.float32),
        pltpu.VMEM((N_GRAPHS, D), jnp.float32),
    ],
)


def kernel(x, edge_index, batch,
           W1_rel, b1_rel, W1_root, W2_rel, b2_rel, W2_root, W_lin, b_lin):
    src = edge_index[0].astype(jnp.int32).reshape(NW * NCHUNK, CHUNK)
    dst = edge_index[1].astype(jnp.int32).reshape(NW * NCHUNK, CHUNK)
    bat = batch.astype(jnp.int32).reshape(NBLK, 1, BLK)
    b1 = b1_rel.reshape(1, D)
    b2 = b2_rel.reshape(1, D)
    bl = b_lin.reshape(1, N_CLASSES)

    p1 = _sc_agg(src, dst, x)
    h1 = _dense1(p1[0], p1[1], x, W1_rel, b1, W1_root)
    p2 = _sc_agg(src, dst, h1)
    out = _dense2(p2[0], p2[1], h1, W2_rel, b2, W2_root, bat, W_lin, bl)
    return out


# speedup vs baseline: 1.0423x; 1.0423x over previous
import functools

import jax
import jax.numpy as jnp
from jax import lax
from jax.experimental import pallas as pl
from jax.experimental.pallas import tpu as pltpu
from jax.experimental.pallas import tpu_sc as plsc

N_NODES = 10000
N_EDGES = 320000
D = 128
N_CLASSES = 10
N_GRAPHS = 64

NC = 2
NS = 16
NW = NC * NS
CHUNK = 50
EPW = N_EDGES // NW
NCHUNK = EPW // CHUNK
NBUF = 4
PHASES = 5
PCH = NCHUNK // PHASES
N_PAD = 10240
RPT = N_PAD // NS
ZROWS = 40
ZCOPIES = RPT // ZROWS
WROWS = 128
WCOPIES = RPT // WROWS

BLK = 1000
NBLK = N_NODES // BLK


def _sc_agg_body(src_hbm, dst_hbm, x_hbm, out_hbm,
                 idx_s, idx_d, buf0, buf1, buf2, buf3, agg_sh,
                 sg0, sg1, sg2, sg3, ss0, ss1, ss2, ss3):
    c = lax.axis_index("c")
    s = lax.axis_index("s")
    wid = s * NC + c
    bufs = (buf0, buf1, buf2, buf3)
    sgs = (sg0, sg1, sg2, sg3)
    sss = (ss0, ss1, ss2, ss3)

    def _zero_row(i, _):
        def _zero_lane(j, _):
            buf0[i, pl.ds(j * 16, 16)] = jnp.zeros((16,), jnp.float32)
            return 0
        return lax.fori_loop(0, D // 16, _zero_lane, 0)
    lax.fori_loop(0, ZROWS, _zero_row, 0)
    zsrc = buf0.at[pl.ds(0, ZROWS)]
    for r in range(ZCOPIES):
        pltpu.sync_copy(zsrc, agg_sh.at[pl.ds(s * RPT + r * ZROWS, ZROWS)])
    plsc.subcore_barrier()

    for p in range(PHASES):
        base = wid * NCHUNK + p * PCH
        pltpu.sync_copy(src_hbm.at[pl.ds(base, PCH)], idx_s)
        pltpu.sync_copy(dst_hbm.at[pl.ds(base, PCH)], idx_d)
        pltpu.async_copy(x_hbm.at[idx_s.at[0]], bufs[0], sgs[0])
        pltpu.async_copy(x_hbm.at[idx_s.at[1]], bufs[1], sgs[1])

        def _round(k, _):
            for b in range(NBUF):
                j = NBUF * k + b
                pltpu.make_async_copy(x_hbm.at[idx_s.at[j]],
                                      bufs[b], sgs[b]).wait()
                pltpu.async_copy(bufs[b], agg_sh.at[idx_d.at[j]],
                                 sss[b], add=True)
                b2 = (b + 2) % NBUF

                @pl.when(j >= 2)
                def _():
                    pltpu.make_async_copy(bufs[b2],
                                          agg_sh.at[idx_d.at[j - 2]],
                                          sss[b2]).wait()

                @pl.when(j + 2 < PCH)
                def _():
                    pltpu.async_copy(x_hbm.at[idx_s.at[j + 2]],
                                     bufs[b2], sgs[b2])
            return 0
        lax.fori_loop(0, PCH // NBUF, _round, 0)
        pltpu.make_async_copy(bufs[2], agg_sh.at[idx_d.at[PCH - 2]],
                              sss[2]).wait()
        pltpu.make_async_copy(bufs[3], agg_sh.at[idx_d.at[PCH - 1]],
                              sss[3]).wait()

    plsc.subcore_barrier()
    for r in range(WCOPIES):
        off = s * RPT + r * WROWS
        pltpu.sync_copy(agg_sh.at[pl.ds(off, WROWS)],
                        out_hbm.at[c, pl.ds(off, WROWS)])


_sc_agg = pl.kernel(
    _sc_agg_body,
    out_type=jax.ShapeDtypeStruct((NC, N_PAD, D), jnp.float32),
    mesh=plsc.VectorSubcoreMesh(core_axis_name="c", subcore_axis_name="s",
                                num_cores=NC, num_subcores=NS),
    scratch_types=[
        pltpu.VMEM((PCH, CHUNK), jnp.int32),
        pltpu.VMEM((PCH, CHUNK), jnp.int32),
        pltpu.VMEM((CHUNK, D), jnp.float32),
        pltpu.VMEM((CHUNK, D), jnp.float32),
        pltpu.VMEM((CHUNK, D), jnp.float32),
        pltpu.VMEM((CHUNK, D), jnp.float32),
        pltpu.VMEM_SHARED((N_PAD, D), jnp.float32),
        pltpu.SemaphoreType.DMA,
        pltpu.SemaphoreType.DMA,
        pltpu.SemaphoreType.DMA,
        pltpu.SemaphoreType.DMA,
        pltpu.SemaphoreType.DMA,
        pltpu.SemaphoreType.DMA,
        pltpu.SemaphoreType.DMA,
        pltpu.SemaphoreType.DMA,
    ],
)


def _dense1_body(a0, a1, x, wrel, b, wroot, h_ref):
    agg = a0[0] + a1[0]
    h = jnp.dot(agg, wrel[...].T, preferred_element_type=jnp.float32)
    h = h + jnp.dot(x[...], wroot[...].T, preferred_element_type=jnp.float32)
    h_ref[...] = jnp.maximum(h + b[...], 0.0)


_dense1 = pl.pallas_call(
    _dense1_body,
    grid=(NBLK,),
    in_specs=[
        pl.BlockSpec((1, BLK, D), lambda i: (0, i, 0)),
        pl.BlockSpec((1, BLK, D), lambda i: (1, i, 0)),
        pl.BlockSpec((BLK, D), lambda i: (i, 0)),
        pl.BlockSpec((D, D), lambda i: (0, 0)),
        pl.BlockSpec((1, D), lambda i: (0, 0)),
        pl.BlockSpec((D, D), lambda i: (0, 0)),
    ],
    out_specs=pl.BlockSpec((BLK, D), lambda i: (i, 0)),
    out_shape=jax.ShapeDtypeStruct((N_NODES, D), jnp.float32),
)


def _dense2_body(a0, a1, h1, wrel, b, wroot, bat, wlin, blin,
                 out_ref, pool_acc, cnt_acc):
    i = pl.program_id(0)
    agg = a0[0] + a1[0]
    h = jnp.dot(agg, wrel[...].T, preferred_element_type=jnp.float32)
    h = h + jnp.dot(h1[...], wroot[...].T, preferred_element_type=jnp.float32)
    h = jnp.maximum(h + b[...], 0.0)

    seg = bat[...].reshape(1, BLK)
    gid = lax.broadcasted_iota(jnp.int32, (N_GRAPHS, BLK), 0)
    onehot = (seg == gid).astype(jnp.float32)

    @pl.when(i == 0)
    def _():
        pool_acc[...] = jnp.zeros_like(pool_acc)
        cnt_acc[...] = jnp.zeros_like(cnt_acc)

    pool_acc[...] += jnp.dot(onehot, h, preferred_element_type=jnp.float32)
    cnt_acc[...] += jnp.sum(onehot, axis=1, keepdims=True)

    @pl.when(i == pl.num_programs(0) - 1)
    def _():
        pooled = pool_acc[...] / jnp.maximum(cnt_acc[...], 1.0)
        out_ref[...] = (jnp.dot(pooled, wlin[...].T,
                                preferred_element_type=jnp.float32)
                        + blin[...])


_dense2 = pl.pallas_call(
    _dense2_body,
    grid=(NBLK,),
    in_specs=[
        pl.BlockSpec((1, BLK, D), lambda i: (0, i, 0)),
        pl.BlockSpec((1, BLK, D), lambda i: (1, i, 0)),
        pl.BlockSpec((BLK, D), lambda i: (i, 0)),
        pl.BlockSpec((D, D), lambda i: (0, 0)),
        pl.BlockSpec((1, D), lambda i: (0, 0)),
        pl.BlockSpec((D, D), lambda i: (0, 0)),
        pl.BlockSpec((1, 1, BLK), lambda i: (i, 0, 0)),
        pl.BlockSpec((N_CLASSES, D), lambda i: (0, 0)),
        pl.BlockSpec((1, N_CLASSES), lambda i: (0, 0)),
    ],
    out_specs=pl.BlockSpec((N_GRAPHS, N_CLASSES), lambda i: (0, 0)),
    out_shape=jax.ShapeDtypeStruct((N_GRAPHS, N_CLASSES), jnp.float32),
    scratch_shapes=[
        pltpu.VMEM((N_GRAPHS, D), jnp.float32),
        pltpu.VMEM((N_GRAPHS, D), jnp.float32),
    ],
)


def kernel(x, edge_index, batch,
           W1_rel, b1_rel, W1_root, W2_rel, b2_rel, W2_root, W_lin, b_lin):
    src = edge_index[0].astype(jnp.int32).reshape(NW * NCHUNK, CHUNK)
    dst = edge_index[1].astype(jnp.int32).reshape(NW * NCHUNK, CHUNK)
    bat = batch.astype(jnp.int32).reshape(NBLK, 1, BLK)
    b1 = b1_rel.reshape(1, D)
    b2 = b2_rel.reshape(1, D)
    bl = b_lin.reshape(1, N_CLASSES)

    p1 = _sc_agg(src, dst, x)
    h1 = _dense1(p1, p1, x, W1_rel, b1, W1_root)
    p2 = _sc_agg(src, dst, h1)
    out = _dense2(p2, p2, h1, W2_rel, b2, W2_root, bat, W_lin, bl)
    return out


# speedup vs baseline: 1.2047x; 1.1558x over previous
import functools

import jax
import jax.numpy as jnp
from jax import lax
from jax.experimental import pallas as pl
from jax.experimental.pallas import tpu as pltpu
from jax.experimental.pallas import tpu_sc as plsc

N_NODES = 10000
N_EDGES = 320000
D = 128
N_CLASSES = 10
N_GRAPHS = 64

NC = 2
NS = 16
NW = NC * NS
CHUNK = 50
EPW = N_EDGES // NW
NCHUNK = EPW // CHUNK
NBUF = 4
PHASES = 5
PCH = NCHUNK // PHASES
N_PAD = 10240
RPT = N_PAD // NS
ZROWS = 40
ZCOPIES = RPT // ZROWS
WROWS = 128
WCOPIES = RPT // WROWS

BLK = 1000
NBLK = N_NODES // BLK


def _sc_agg_body(src_hbm, dst_hbm, x_hbm, out_hbm,
                 idx_s, idx_d, buf0, buf1, buf2, buf3, agg_sh,
                 sg0, sg1, sg2, sg3, ss0, ss1, ss2, ss3):
    c = lax.axis_index("c")
    s = lax.axis_index("s")
    wid = s * NC + c
    bufs = (buf0, buf1, buf2, buf3)
    sgs = (sg0, sg1, sg2, sg3)
    sss = (ss0, ss1, ss2, ss3)

    def _zero_row(i, _):
        def _zero_lane(j, _):
            buf0[i, pl.ds(j * 16, 16)] = jnp.zeros((16,), jnp.float32)
            return 0
        return lax.fori_loop(0, D // 16, _zero_lane, 0)
    lax.fori_loop(0, ZROWS, _zero_row, 0)
    zsrc = buf0.at[pl.ds(0, ZROWS)]
    for r in range(ZCOPIES):
        pltpu.sync_copy(zsrc, agg_sh.at[pl.ds(s * RPT + r * ZROWS, ZROWS)])
    plsc.subcore_barrier()

    for p in range(PHASES):
        base = wid * NCHUNK + p * PCH
        pltpu.sync_copy(src_hbm.at[pl.ds(base, PCH)], idx_s)
        pltpu.sync_copy(dst_hbm.at[pl.ds(base, PCH)], idx_d)
        pltpu.async_copy(x_hbm.at[idx_s.at[0]], bufs[0], sgs[0])
        pltpu.async_copy(x_hbm.at[idx_s.at[1]], bufs[1], sgs[1])
        pltpu.async_copy(x_hbm.at[idx_s.at[2]], bufs[2], sgs[2])

        def _round(k, _):
            for b in range(NBUF):
                j = NBUF * k + b
                pltpu.make_async_copy(x_hbm.at[idx_s.at[j]],
                                      bufs[b], sgs[b]).wait()
                pltpu.async_copy(bufs[b], agg_sh.at[idx_d.at[j]],
                                 sss[b], add=True)
                b2 = (b + 3) % NBUF

                @pl.when(j >= 1)
                def _():
                    pltpu.make_async_copy(bufs[b2],
                                          agg_sh.at[idx_d.at[j - 1]],
                                          sss[b2]).wait()

                @pl.when(j + 3 < PCH)
                def _():
                    pltpu.async_copy(x_hbm.at[idx_s.at[j + 3]],
                                     bufs[b2], sgs[b2])
            return 0
        lax.fori_loop(0, PCH // NBUF, _round, 0)
        pltpu.make_async_copy(bufs[3], agg_sh.at[idx_d.at[PCH - 1]],
                              sss[3]).wait()

    plsc.subcore_barrier()
    for r in range(WCOPIES):
        off = s * RPT + r * WROWS
        pltpu.sync_copy(agg_sh.at[pl.ds(off, WROWS)],
                        out_hbm.at[c, pl.ds(off, WROWS)])


_sc_agg = pl.kernel(
    _sc_agg_body,
    out_type=jax.ShapeDtypeStruct((NC, N_PAD, D), jnp.float32),
    mesh=plsc.VectorSubcoreMesh(core_axis_name="c", subcore_axis_name="s",
                                num_cores=NC, num_subcores=NS),
    scratch_types=[
        pltpu.VMEM((PCH, CHUNK), jnp.int32),
        pltpu.VMEM((PCH, CHUNK), jnp.int32),
        pltpu.VMEM((CHUNK, D), jnp.float32),
        pltpu.VMEM((CHUNK, D), jnp.float32),
        pltpu.VMEM((CHUNK, D), jnp.float32),
        pltpu.VMEM((CHUNK, D), jnp.float32),
        pltpu.VMEM_SHARED((N_PAD, D), jnp.float32),
        pltpu.SemaphoreType.DMA,
        pltpu.SemaphoreType.DMA,
        pltpu.SemaphoreType.DMA,
        pltpu.SemaphoreType.DMA,
        pltpu.SemaphoreType.DMA,
        pltpu.SemaphoreType.DMA,
        pltpu.SemaphoreType.DMA,
        pltpu.SemaphoreType.DMA,
    ],
)


def _dense1_body(a0, a1, x, wrel, b, wroot, h_ref):
    agg = a0[0] + a1[0]
    h = jnp.dot(agg, wrel[...].T, preferred_element_type=jnp.float32)
    h = h + jnp.dot(x[...], wroot[...].T, preferred_element_type=jnp.float32)
    h_ref[...] = jnp.maximum(h + b[...], 0.0)


_dense1 = pl.pallas_call(
    _dense1_body,
    grid=(NBLK,),
    in_specs=[
        pl.BlockSpec((1, BLK, D), lambda i: (0, i, 0)),
        pl.BlockSpec((1, BLK, D), lambda i: (1, i, 0)),
        pl.BlockSpec((BLK, D), lambda i: (i, 0)),
        pl.BlockSpec((D, D), lambda i: (0, 0)),
        pl.BlockSpec((1, D), lambda i: (0, 0)),
        pl.BlockSpec((D, D), lambda i: (0, 0)),
    ],
    out_specs=pl.BlockSpec((BLK, D), lambda i: (i, 0)),
    out_shape=jax.ShapeDtypeStruct((N_NODES, D), jnp.float32),
)


def _dense2_body(a0, a1, h1, wrel, b, wroot, bat, wlin, blin,
                 out_ref, pool_acc, cnt_acc):
    i = pl.program_id(0)
    agg = a0[0] + a1[0]
    h = jnp.dot(agg, wrel[...].T, preferred_element_type=jnp.float32)
    h = h + jnp.dot(h1[...], wroot[...].T, preferred_element_type=jnp.float32)
    h = jnp.maximum(h + b[...], 0.0)

    seg = bat[...].reshape(1, BLK)
    gid = lax.broadcasted_iota(jnp.int32, (N_GRAPHS, BLK), 0)
    onehot = (seg == gid).astype(jnp.float32)

    @pl.when(i == 0)
    def _():
        pool_acc[...] = jnp.zeros_like(pool_acc)
        cnt_acc[...] = jnp.zeros_like(cnt_acc)

    pool_acc[...] += jnp.dot(onehot, h, preferred_element_type=jnp.float32)
    cnt_acc[...] += jnp.sum(onehot, axis=1, keepdims=True)

    @pl.when(i == pl.num_programs(0) - 1)
    def _():
        pooled = pool_acc[...] / jnp.maximum(cnt_acc[...], 1.0)
        out_ref[...] = (jnp.dot(pooled, wlin[...].T,
                                preferred_element_type=jnp.float32)
                        + blin[...])


_dense2 = pl.pallas_call(
    _dense2_body,
    grid=(NBLK,),
    in_specs=[
        pl.BlockSpec((1, BLK, D), lambda i: (0, i, 0)),
        pl.BlockSpec((1, BLK, D), lambda i: (1, i, 0)),
        pl.BlockSpec((BLK, D), lambda i: (i, 0)),
        pl.BlockSpec((D, D), lambda i: (0, 0)),
        pl.BlockSpec((1, D), lambda i: (0, 0)),
        pl.BlockSpec((D, D), lambda i: (0, 0)),
        pl.BlockSpec((1, 1, BLK), lambda i: (i, 0, 0)),
        pl.BlockSpec((N_CLASSES, D), lambda i: (0, 0)),
        pl.BlockSpec((1, N_CLASSES), lambda i: (0, 0)),
    ],
    out_specs=pl.BlockSpec((N_GRAPHS, N_CLASSES), lambda i: (0, 0)),
    out_shape=jax.ShapeDtypeStruct((N_GRAPHS, N_CLASSES), jnp.float32),
    scratch_shapes=[
        pltpu.VMEM((N_GRAPHS, D), jnp.float32),
        pltpu.VMEM((N_GRAPHS, D), jnp.float32),
    ],
)


def kernel(x, edge_index, batch,
           W1_rel, b1_rel, W1_root, W2_rel, b2_rel, W2_root, W_lin, b_lin):
    src = edge_index[0].astype(jnp.int32).reshape(NW * NCHUNK, CHUNK)
    dst = edge_index[1].astype(jnp.int32).reshape(NW * NCHUNK, CHUNK)
    bat = batch.astype(jnp.int32).reshape(NBLK, 1, BLK)
    b1 = b1_rel.reshape(1, D)
    b2 = b2_rel.reshape(1, D)
    bl = b_lin.reshape(1, N_CLASSES)

    p1 = _sc_agg(src, dst, x)
    h1 = _dense1(p1, p1, x, W1_rel, b1, W1_root)
    p2 = _sc_agg(src, dst, h1)
    out = _dense2(p2, p2, h1, W2_rel, b2, W2_root, bat, W_lin, bl)
    return out


# speedup vs baseline: 1.2561x; 1.0427x over previous
import functools

import jax
import jax.numpy as jnp
from jax import lax
from jax.experimental import pallas as pl
from jax.experimental.pallas import tpu as pltpu
from jax.experimental.pallas import tpu_sc as plsc

N_NODES = 10000
N_EDGES = 320000
D = 128
N_CLASSES = 10
N_GRAPHS = 64

NC = 2
NS = 16
NW = NC * NS
CHUNK = 50
EPW = N_EDGES // NW
NCHUNK = EPW // CHUNK
NBUF = 5
PHASES = 5
PCH = NCHUNK // PHASES
N_PAD = 10240
RPT = N_PAD // NS
ZROWS = 40
ZCOPIES = RPT // ZROWS
WROWS = 128
WCOPIES = RPT // WROWS

BLK = 1000
NBLK = N_NODES // BLK


def _sc_agg_body(src_hbm, dst_hbm, x_hbm, out_hbm,
                 idx_s, idx_d, buf0, buf1, buf2, buf3, buf4, agg_sh,
                 sg0, sg1, sg2, sg3, sg4, ss0, ss1, ss2, ss3, ss4):
    c = lax.axis_index("c")
    s = lax.axis_index("s")
    wid = s * NC + c
    bufs = (buf0, buf1, buf2, buf3, buf4)
    sgs = (sg0, sg1, sg2, sg3, sg4)
    sss = (ss0, ss1, ss2, ss3, ss4)

    def _zero_row(i, _):
        def _zero_lane(j, _):
            buf0[i, pl.ds(j * 16, 16)] = jnp.zeros((16,), jnp.float32)
            return 0
        return lax.fori_loop(0, D // 16, _zero_lane, 0)
    lax.fori_loop(0, ZROWS, _zero_row, 0)
    zsrc = buf0.at[pl.ds(0, ZROWS)]
    for r in range(ZCOPIES):
        pltpu.sync_copy(zsrc, agg_sh.at[pl.ds(s * RPT + r * ZROWS, ZROWS)])
    plsc.subcore_barrier()

    for p in range(PHASES):
        base = wid * NCHUNK + p * PCH
        pltpu.sync_copy(src_hbm.at[pl.ds(base, PCH)], idx_s)
        pltpu.sync_copy(dst_hbm.at[pl.ds(base, PCH)], idx_d)
        for b in range(NBUF - 1):
            pltpu.async_copy(x_hbm.at[idx_s.at[b]], bufs[b], sgs[b])

        def _round(k, _):
            for b in range(NBUF):
                j = NBUF * k + b
                pltpu.make_async_copy(x_hbm.at[idx_s.at[j]],
                                      bufs[b], sgs[b]).wait()
                pltpu.async_copy(bufs[b], agg_sh.at[idx_d.at[j]],
                                 sss[b], add=True)
                b2 = (b + NBUF - 1) % NBUF

                @pl.when(j >= 1)
                def _():
                    pltpu.make_async_copy(bufs[b2],
                                          agg_sh.at[idx_d.at[j - 1]],
                                          sss[b2]).wait()

                @pl.when(j + NBUF - 1 < PCH)
                def _():
                    pltpu.async_copy(x_hbm.at[idx_s.at[j + NBUF - 1]],
                                     bufs[b2], sgs[b2])
            return 0
        lax.fori_loop(0, PCH // NBUF, _round, 0)
        pltpu.make_async_copy(bufs[(PCH - 1) % NBUF],
                              agg_sh.at[idx_d.at[PCH - 1]],
                              sss[(PCH - 1) % NBUF]).wait()

    plsc.subcore_barrier()
    for r in range(WCOPIES):
        off = s * RPT + r * WROWS
        pltpu.sync_copy(agg_sh.at[pl.ds(off, WROWS)],
                        out_hbm.at[c, pl.ds(off, WROWS)])


_sc_agg = pl.kernel(
    _sc_agg_body,
    out_type=jax.ShapeDtypeStruct((NC, N_PAD, D), jnp.float32),
    mesh=plsc.VectorSubcoreMesh(core_axis_name="c", subcore_axis_name="s",
                                num_cores=NC, num_subcores=NS),
    scratch_types=[
        pltpu.VMEM((PCH, CHUNK), jnp.int32),
        pltpu.VMEM((PCH, CHUNK), jnp.int32),
        pltpu.VMEM((CHUNK, D), jnp.float32),
        pltpu.VMEM((CHUNK, D), jnp.float32),
        pltpu.VMEM((CHUNK, D), jnp.float32),
        pltpu.VMEM((CHUNK, D), jnp.float32),
        pltpu.VMEM((CHUNK, D), jnp.float32),
        pltpu.VMEM_SHARED((N_PAD, D), jnp.float32),
        pltpu.SemaphoreType.DMA,
        pltpu.SemaphoreType.DMA,
        pltpu.SemaphoreType.DMA,
        pltpu.SemaphoreType.DMA,
        pltpu.SemaphoreType.DMA,
        pltpu.SemaphoreType.DMA,
        pltpu.SemaphoreType.DMA,
        pltpu.SemaphoreType.DMA,
        pltpu.SemaphoreType.DMA,
        pltpu.SemaphoreType.DMA,
    ],
)


def _dense1_body(a0, a1, x, wrel, b, wroot, h_ref):
    agg = a0[0] + a1[0]
    h = jnp.dot(agg, wrel[...].T, preferred_element_type=jnp.float32)
    h = h + jnp.dot(x[...], wroot[...].T, preferred_element_type=jnp.float32)
    h_ref[...] = jnp.maximum(h + b[...], 0.0)


_dense1 = pl.pallas_call(
    _dense1_body,
    grid=(NBLK,),
    in_specs=[
        pl.BlockSpec((1, BLK, D), lambda i: (0, i, 0)),
        pl.BlockSpec((1, BLK, D), lambda i: (1, i, 0)),
        pl.BlockSpec((BLK, D), lambda i: (i, 0)),
        pl.BlockSpec((D, D), lambda i: (0, 0)),
        pl.BlockSpec((1, D), lambda i: (0, 0)),
        pl.BlockSpec((D, D), lambda i: (0, 0)),
    ],
    out_specs=pl.BlockSpec((BLK, D), lambda i: (i, 0)),
    out_shape=jax.ShapeDtypeStruct((N_NODES, D), jnp.float32),
)


def _dense2_body(a0, a1, h1, wrel, b, wroot, bat, wlin, blin,
                 out_ref, pool_acc, cnt_acc):
    i = pl.program_id(0)
    agg = a0[0] + a1[0]
    h = jnp.dot(agg, wrel[...].T, preferred_element_type=jnp.float32)
    h = h + jnp.dot(h1[...], wroot[...].T, preferred_element_type=jnp.float32)
    h = jnp.maximum(h + b[...], 0.0)

    seg = bat[...].reshape(1, BLK)
    gid = lax.broadcasted_iota(jnp.int32, (N_GRAPHS, BLK), 0)
    onehot = (seg == gid).astype(jnp.float32)

    @pl.when(i == 0)
    def _():
        pool_acc[...] = jnp.zeros_like(pool_acc)
        cnt_acc[...] = jnp.zeros_like(cnt_acc)

    pool_acc[...] += jnp.dot(onehot, h, preferred_element_type=jnp.float32)
    cnt_acc[...] += jnp.sum(onehot, axis=1, keepdims=True)

    @pl.when(i == pl.num_programs(0) - 1)
    def _():
        pooled = pool_acc[...] / jnp.maximum(cnt_acc[...], 1.0)
        out_ref[...] = (jnp.dot(pooled, wlin[...].T,
                                preferred_element_type=jnp.float32)
                        + blin[...])


_dense2 = pl.pallas_call(
    _dense2_body,
    grid=(NBLK,),
    in_specs=[
        pl.BlockSpec((1, BLK, D), lambda i: (0, i, 0)),
        pl.BlockSpec((1, BLK, D), lambda i: (1, i, 0)),
        pl.BlockSpec((BLK, D), lambda i: (i, 0)),
        pl.BlockSpec((D, D), lambda i: (0, 0)),
        pl.BlockSpec((1, D), lambda i: (0, 0)),
        pl.BlockSpec((D, D), lambda i: (0, 0)),
        pl.BlockSpec((1, 1, BLK), lambda i: (i, 0, 0)),
        pl.BlockSpec((N_CLASSES, D), lambda i: (0, 0)),
        pl.BlockSpec((1, N_CLASSES), lambda i: (0, 0)),
    ],
    out_specs=pl.BlockSpec((N_GRAPHS, N_CLASSES), lambda i: (0, 0)),
    out_shape=jax.ShapeDtypeStruct((N_GRAPHS, N_CLASSES), jnp.float32),
    scratch_shapes=[
        pltpu.VMEM((N_GRAPHS, D), jnp.float32),
        pltpu.VMEM((N_GRAPHS, D), jnp.float32),
    ],
)


def kernel(x, edge_index, batch,
           W1_rel, b1_rel, W1_root, W2_rel, b2_rel, W2_root, W_lin, b_lin):
    src = edge_index[0].astype(jnp.int32).reshape(NW * NCHUNK, CHUNK)
    dst = edge_index[1].astype(jnp.int32).reshape(NW * NCHUNK, CHUNK)
    bat = batch.astype(jnp.int32).reshape(NBLK, 1, BLK)
    b1 = b1_rel.reshape(1, D)
    b2 = b2_rel.reshape(1, D)
    bl = b_lin.reshape(1, N_CLASSES)

    p1 = _sc_agg(src, dst, x)
    h1 = _dense1(p1, p1, x, W1_rel, b1, W1_root)
    p2 = _sc_agg(src, dst, h1)
    out = _dense2(p2, p2, h1, W2_rel, b2, W2_root, bat, W_lin, bl)
    return out


# speedup vs baseline: 1.2850x; 1.0230x over previous
import functools

import jax
import jax.numpy as jnp
from jax import lax
from jax.experimental import pallas as pl
from jax.experimental.pallas import tpu as pltpu
from jax.experimental.pallas import tpu_sc as plsc

N_NODES = 10000
N_EDGES = 320000
D = 128
N_CLASSES = 10
N_GRAPHS = 64

NC = 2
NS = 16
NW = NC * NS
CHUNK = 50
EPW = N_EDGES // NW
NCHUNK = EPW // CHUNK
NBUF = 5
PHASES = 5
PCH = NCHUNK // PHASES
N_PAD = 10240
RPT = N_PAD // NS
ZROWS = 40
ZCOPIES = RPT // ZROWS
WROWS = 128
WCOPIES = RPT // WROWS

BLK = 2000
NBLK = N_NODES // BLK


def _sc_agg_body(src_hbm, dst_hbm, x_hbm, out_hbm,
                 idx_s, idx_d, buf0, buf1, buf2, buf3, buf4, agg_sh,
                 sg0, sg1, sg2, sg3, sg4, ss0, ss1, ss2, ss3, ss4):
    c = lax.axis_index("c")
    s = lax.axis_index("s")
    wid = s * NC + c
    bufs = (buf0, buf1, buf2, buf3, buf4)
    sgs = (sg0, sg1, sg2, sg3, sg4)
    sss = (ss0, ss1, ss2, ss3, ss4)

    def _zero_row(i, _):
        def _zero_lane(j, _):
            buf0[i, pl.ds(j * 16, 16)] = jnp.zeros((16,), jnp.float32)
            return 0
        return lax.fori_loop(0, D // 16, _zero_lane, 0)
    lax.fori_loop(0, ZROWS, _zero_row, 0)
    zsrc = buf0.at[pl.ds(0, ZROWS)]
    for r in range(ZCOPIES):
        pltpu.sync_copy(zsrc, agg_sh.at[pl.ds(s * RPT + r * ZROWS, ZROWS)])
    plsc.subcore_barrier()

    for p in range(PHASES):
        base = wid * NCHUNK + p * PCH
        pltpu.sync_copy(src_hbm.at[pl.ds(base, PCH)], idx_s)
        pltpu.sync_copy(dst_hbm.at[pl.ds(base, PCH)], idx_d)
        for b in range(NBUF - 1):
            pltpu.async_copy(x_hbm.at[idx_s.at[b]], bufs[b], sgs[b])

        def _round(k, _):
            for b in range(NBUF):
                j = NBUF * k + b
                pltpu.make_async_copy(x_hbm.at[idx_s.at[j]],
                                      bufs[b], sgs[b]).wait()
                pltpu.async_copy(bufs[b], agg_sh.at[idx_d.at[j]],
                                 sss[b], add=True)
                b2 = (b + NBUF - 1) % NBUF

                @pl.when(j >= 1)
                def _():
                    pltpu.make_async_copy(bufs[b2],
                                          agg_sh.at[idx_d.at[j - 1]],
                                          sss[b2]).wait()

                @pl.when(j + NBUF - 1 < PCH)
                def _():
                    pltpu.async_copy(x_hbm.at[idx_s.at[j + NBUF - 1]],
                                     bufs[b2], sgs[b2])
            return 0
        lax.fori_loop(0, PCH // NBUF, _round, 0)
        pltpu.make_async_copy(bufs[(PCH - 1) % NBUF],
                              agg_sh.at[idx_d.at[PCH - 1]],
                              sss[(PCH - 1) % NBUF]).wait()

    plsc.subcore_barrier()
    for r in range(WCOPIES):
        off = s * RPT + r * WROWS
        pltpu.sync_copy(agg_sh.at[pl.ds(off, WROWS)],
                        out_hbm.at[c, pl.ds(off, WROWS)])


_sc_agg = pl.kernel(
    _sc_agg_body,
    out_type=jax.ShapeDtypeStruct((NC, N_PAD, D), jnp.float32),
    mesh=plsc.VectorSubcoreMesh(core_axis_name="c", subcore_axis_name="s",
                                num_cores=NC, num_subcores=NS),
    scratch_types=[
        pltpu.VMEM((PCH, CHUNK), jnp.int32),
        pltpu.VMEM((PCH, CHUNK), jnp.int32),
        pltpu.VMEM((CHUNK, D), jnp.float32),
        pltpu.VMEM((CHUNK, D), jnp.float32),
        pltpu.VMEM((CHUNK, D), jnp.float32),
        pltpu.VMEM((CHUNK, D), jnp.float32),
        pltpu.VMEM((CHUNK, D), jnp.float32),
        pltpu.VMEM_SHARED((N_PAD, D), jnp.float32),
        pltpu.SemaphoreType.DMA,
        pltpu.SemaphoreType.DMA,
        pltpu.SemaphoreType.DMA,
        pltpu.SemaphoreType.DMA,
        pltpu.SemaphoreType.DMA,
        pltpu.SemaphoreType.DMA,
        pltpu.SemaphoreType.DMA,
        pltpu.SemaphoreType.DMA,
        pltpu.SemaphoreType.DMA,
        pltpu.SemaphoreType.DMA,
    ],
)


def _dense1_body(a0, a1, x, wrel, b, wroot, h_ref):
    agg = a0[0] + a1[0]
    h = jnp.dot(agg, wrel[...].T, preferred_element_type=jnp.float32)
    h = h + jnp.dot(x[...], wroot[...].T, preferred_element_type=jnp.float32)
    h_ref[...] = jnp.maximum(h + b[...], 0.0)


_dense1 = pl.pallas_call(
    _dense1_body,
    grid=(NBLK,),
    in_specs=[
        pl.BlockSpec((1, BLK, D), lambda i: (0, i, 0)),
        pl.BlockSpec((1, BLK, D), lambda i: (1, i, 0)),
        pl.BlockSpec((BLK, D), lambda i: (i, 0)),
        pl.BlockSpec((D, D), lambda i: (0, 0)),
        pl.BlockSpec((1, D), lambda i: (0, 0)),
        pl.BlockSpec((D, D), lambda i: (0, 0)),
    ],
    out_specs=pl.BlockSpec((BLK, D), lambda i: (i, 0)),
    out_shape=jax.ShapeDtypeStruct((N_NODES, D), jnp.float32),
)


def _dense2_body(a0, a1, h1, wrel, b, wroot, bat, wlin, blin,
                 out_ref, pool_acc, cnt_acc):
    i = pl.program_id(0)
    agg = a0[0] + a1[0]
    h = jnp.dot(agg, wrel[...].T, preferred_element_type=jnp.float32)
    h = h + jnp.dot(h1[...], wroot[...].T, preferred_element_type=jnp.float32)
    h = jnp.maximum(h + b[...], 0.0)

    seg = bat[...].reshape(1, BLK)
    gid = lax.broadcasted_iota(jnp.int32, (N_GRAPHS, BLK), 0)
    onehot = (seg == gid).astype(jnp.float32)

    @pl.when(i == 0)
    def _():
        pool_acc[...] = jnp.zeros_like(pool_acc)
        cnt_acc[...] = jnp.zeros_like(cnt_acc)

    pool_acc[...] += jnp.dot(onehot, h, preferred_element_type=jnp.float32)
    cnt_acc[...] += jnp.sum(onehot, axis=1, keepdims=True)

    @pl.when(i == pl.num_programs(0) - 1)
    def _():
        pooled = pool_acc[...] / jnp.maximum(cnt_acc[...], 1.0)
        out_ref[...] = (jnp.dot(pooled, wlin[...].T,
                                preferred_element_type=jnp.float32)
                        + blin[...])


_dense2 = pl.pallas_call(
    _dense2_body,
    grid=(NBLK,),
    in_specs=[
        pl.BlockSpec((1, BLK, D), lambda i: (0, i, 0)),
        pl.BlockSpec((1, BLK, D), lambda i: (1, i, 0)),
        pl.BlockSpec((BLK, D), lambda i: (i, 0)),
        pl.BlockSpec((D, D), lambda i: (0, 0)),
        pl.BlockSpec((1, D), lambda i: (0, 0)),
        pl.BlockSpec((D, D), lambda i: (0, 0)),
        pl.BlockSpec((1, 1, BLK), lambda i: (i, 0, 0)),
        pl.BlockSpec((N_CLASSES, D), lambda i: (0, 0)),
        pl.BlockSpec((1, N_CLASSES), lambda i: (0, 0)),
    ],
    out_specs=pl.BlockSpec((N_GRAPHS, N_CLASSES), lambda i: (0, 0)),
    out_shape=jax.ShapeDtypeStruct((N_GRAPHS, N_CLASSES), jnp.float32),
    scratch_shapes=[
        pltpu.VMEM((N_GRAPHS, D), jnp.float32),
        pltpu.VMEM((N_GRAPHS, D), jnp.float32),
    ],
)


def kernel(x, edge_index, batch,
           W1_rel, b1_rel, W1_root, W2_rel, b2_rel, W2_root, W_lin, b_lin):
    src = edge_index[0].astype(jnp.int32).reshape(NW * NCHUNK, CHUNK)
    dst = edge_index[1].astype(jnp.int32).reshape(NW * NCHUNK, CHUNK)
    bat = batch.astype(jnp.int32).reshape(NBLK, 1, BLK)
    b1 = b1_rel.reshape(1, D)
    b2 = b2_rel.reshape(1, D)
    bl = b_lin.reshape(1, N_CLASSES)

    p1 = _sc_agg(src, dst, x)
    h1 = _dense1(p1, p1, x, W1_rel, b1, W1_root)
    p2 = _sc_agg(src, dst, h1)
    out = _dense2(p2, p2, h1, W2_rel, b2, W2_root, bat, W_lin, bl)
    return out


# speedup vs baseline: 1.2962x; 1.0088x over previous
import functools

import jax
import jax.numpy as jnp
from jax import lax
from jax.experimental import pallas as pl
from jax.experimental.pallas import tpu as pltpu
from jax.experimental.pallas import tpu_sc as plsc

N_NODES = 10000
N_EDGES = 320000
D = 128
N_CLASSES = 10
N_GRAPHS = 64

NC = 2
NS = 16
NW = NC * NS
CHUNK = 50
EPW = N_EDGES // NW
NCHUNK = EPW // CHUNK
NBUF = 5
PHASES = 5
PCH = NCHUNK // PHASES
N_PAD = 10240
RPT = N_PAD // NS
ZROWS = 40
ZCOPIES = RPT // ZROWS
WROWS = 128
WCOPIES = RPT // WROWS

BLK = 5000
NBLK = N_NODES // BLK


def _sc_agg_body(src_hbm, dst_hbm, x_hbm, out_hbm,
                 idx_s, idx_d, buf0, buf1, buf2, buf3, buf4, agg_sh,
                 sg0, sg1, sg2, sg3, sg4, ss0, ss1, ss2, ss3, ss4):
    c = lax.axis_index("c")
    s = lax.axis_index("s")
    wid = s * NC + c
    bufs = (buf0, buf1, buf2, buf3, buf4)
    sgs = (sg0, sg1, sg2, sg3, sg4)
    sss = (ss0, ss1, ss2, ss3, ss4)

    def _zero_row(i, _):
        def _zero_lane(j, _):
            buf0[i, pl.ds(j * 16, 16)] = jnp.zeros((16,), jnp.float32)
            return 0
        return lax.fori_loop(0, D // 16, _zero_lane, 0)
    lax.fori_loop(0, ZROWS, _zero_row, 0)
    zsrc = buf0.at[pl.ds(0, ZROWS)]
    for r in range(ZCOPIES):
        pltpu.sync_copy(zsrc, agg_sh.at[pl.ds(s * RPT + r * ZROWS, ZROWS)])
    plsc.subcore_barrier()

    for p in range(PHASES):
        base = wid * NCHUNK + p * PCH
        pltpu.sync_copy(src_hbm.at[pl.ds(base, PCH)], idx_s)
        pltpu.sync_copy(dst_hbm.at[pl.ds(base, PCH)], idx_d)
        for b in range(NBUF - 1):
            pltpu.async_copy(x_hbm.at[idx_s.at[b]], bufs[b], sgs[b])

        def _round(k, _):
            for b in range(NBUF):
                j = NBUF * k + b
                pltpu.make_async_copy(x_hbm.at[idx_s.at[j]],
                                      bufs[b], sgs[b]).wait()
                pltpu.async_copy(bufs[b], agg_sh.at[idx_d.at[j]],
                                 sss[b], add=True)
                b2 = (b + NBUF - 1) % NBUF

                @pl.when(j >= 1)
                def _():
                    pltpu.make_async_copy(bufs[b2],
                                          agg_sh.at[idx_d.at[j - 1]],
                                          sss[b2]).wait()

                @pl.when(j + NBUF - 1 < PCH)
                def _():
                    pltpu.async_copy(x_hbm.at[idx_s.at[j + NBUF - 1]],
                                     bufs[b2], sgs[b2])
            return 0
        lax.fori_loop(0, PCH // NBUF, _round, 0)
        pltpu.make_async_copy(bufs[(PCH - 1) % NBUF],
                              agg_sh.at[idx_d.at[PCH - 1]],
                              sss[(PCH - 1) % NBUF]).wait()

    plsc.subcore_barrier()
    for r in range(WCOPIES):
        off = s * RPT + r * WROWS
        pltpu.sync_copy(agg_sh.at[pl.ds(off, WROWS)],
                        out_hbm.at[c, pl.ds(off, WROWS)])


_sc_agg = pl.kernel(
    _sc_agg_body,
    out_type=jax.ShapeDtypeStruct((NC, N_PAD, D), jnp.float32),
    mesh=plsc.VectorSubcoreMesh(core_axis_name="c", subcore_axis_name="s",
                                num_cores=NC, num_subcores=NS),
    scratch_types=[
        pltpu.VMEM((PCH, CHUNK), jnp.int32),
        pltpu.VMEM((PCH, CHUNK), jnp.int32),
        pltpu.VMEM((CHUNK, D), jnp.float32),
        pltpu.VMEM((CHUNK, D), jnp.float32),
        pltpu.VMEM((CHUNK, D), jnp.float32),
        pltpu.VMEM((CHUNK, D), jnp.float32),
        pltpu.VMEM((CHUNK, D), jnp.float32),
        pltpu.VMEM_SHARED((N_PAD, D), jnp.float32),
        pltpu.SemaphoreType.DMA,
        pltpu.SemaphoreType.DMA,
        pltpu.SemaphoreType.DMA,
        pltpu.SemaphoreType.DMA,
        pltpu.SemaphoreType.DMA,
        pltpu.SemaphoreType.DMA,
        pltpu.SemaphoreType.DMA,
        pltpu.SemaphoreType.DMA,
        pltpu.SemaphoreType.DMA,
        pltpu.SemaphoreType.DMA,
    ],
)


def _dense1_body(a0, a1, x, wrel, b, wroot, h_ref):
    agg = a0[0] + a1[0]
    h = jnp.dot(agg, wrel[...].T, preferred_element_type=jnp.float32)
    h = h + jnp.dot(x[...], wroot[...].T, preferred_element_type=jnp.float32)
    h_ref[...] = jnp.maximum(h + b[...], 0.0)


_dense1 = pl.pallas_call(
    _dense1_body,
    grid=(NBLK,),
    in_specs=[
        pl.BlockSpec((1, BLK, D), lambda i: (0, i, 0)),
        pl.BlockSpec((1, BLK, D), lambda i: (1, i, 0)),
        pl.BlockSpec((BLK, D), lambda i: (i, 0)),
        pl.BlockSpec((D, D), lambda i: (0, 0)),
        pl.BlockSpec((1, D), lambda i: (0, 0)),
        pl.BlockSpec((D, D), lambda i: (0, 0)),
    ],
    out_specs=pl.BlockSpec((BLK, D), lambda i: (i, 0)),
    out_shape=jax.ShapeDtypeStruct((N_NODES, D), jnp.float32),
)


def _dense2_body(a0, a1, h1, wrel, b, wroot, bat, wlin, blin,
                 out_ref, pool_acc, cnt_acc):
    i = pl.program_id(0)
    agg = a0[0] + a1[0]
    h = jnp.dot(agg, wrel[...].T, preferred_element_type=jnp.float32)
    h = h + jnp.dot(h1[...], wroot[...].T, preferred_element_type=jnp.float32)
    h = jnp.maximum(h + b[...], 0.0)

    seg = bat[...].reshape(1, BLK)
    gid = lax.broadcasted_iota(jnp.int32, (N_GRAPHS, BLK), 0)
    onehot = (seg == gid).astype(jnp.float32)

    @pl.when(i == 0)
    def _():
        pool_acc[...] = jnp.zeros_like(pool_acc)
        cnt_acc[...] = jnp.zeros_like(cnt_acc)

    pool_acc[...] += jnp.dot(onehot, h, preferred_element_type=jnp.float32)
    cnt_acc[...] += jnp.sum(onehot, axis=1, keepdims=True)

    @pl.when(i == pl.num_programs(0) - 1)
    def _():
        pooled = pool_acc[...] / jnp.maximum(cnt_acc[...], 1.0)
        out_ref[...] = (jnp.dot(pooled, wlin[...].T,
                                preferred_element_type=jnp.float32)
                        + blin[...])


_dense2 = pl.pallas_call(
    _dense2_body,
    grid=(NBLK,),
    in_specs=[
        pl.BlockSpec((1, BLK, D), lambda i: (0, i, 0)),
        pl.BlockSpec((1, BLK, D), lambda i: (1, i, 0)),
        pl.BlockSpec((BLK, D), lambda i: (i, 0)),
        pl.BlockSpec((D, D), lambda i: (0, 0)),
        pl.BlockSpec((1, D), lambda i: (0, 0)),
        pl.BlockSpec((D, D), lambda i: (0, 0)),
        pl.BlockSpec((1, 1, BLK), lambda i: (i, 0, 0)),
        pl.BlockSpec((N_CLASSES, D), lambda i: (0, 0)),
        pl.BlockSpec((1, N_CLASSES), lambda i: (0, 0)),
    ],
    out_specs=pl.BlockSpec((N_GRAPHS, N_CLASSES), lambda i: (0, 0)),
    out_shape=jax.ShapeDtypeStruct((N_GRAPHS, N_CLASSES), jnp.float32),
    scratch_shapes=[
        pltpu.VMEM((N_GRAPHS, D), jnp.float32),
        pltpu.VMEM((N_GRAPHS, D), jnp.float32),
    ],
)


def kernel(x, edge_index, batch,
           W1_rel, b1_rel, W1_root, W2_rel, b2_rel, W2_root, W_lin, b_lin):
    src = edge_index[0].astype(jnp.int32).reshape(NW * NCHUNK, CHUNK)
    dst = edge_index[1].astype(jnp.int32).reshape(NW * NCHUNK, CHUNK)
    bat = batch.astype(jnp.int32).reshape(NBLK, 1, BLK)
    b1 = b1_rel.reshape(1, D)
    b2 = b2_rel.reshape(1, D)
    bl = b_lin.reshape(1, N_CLASSES)

    p1 = _sc_agg(src, dst, x)
    h1 = _dense1(p1, p1, x, W1_rel, b1, W1_root)
    p2 = _sc_agg(src, dst, h1)
    out = _dense2(p2, p2, h1, W2_rel, b2, W2_root, bat, W_lin, bl)
    return out
